# baseline (speedup 1.0000x reference)
import functools

import jax
import jax.numpy as jnp
from jax import lax
from jax.experimental import pallas as pl
from jax.experimental.pallas import tpu as pltpu
from jax.experimental.pallas import tpu_sc as plsc

BATCH = 4096
HIST = 200
D = 64
DP = 128
V = 1000000
NC, NS = 2, 16
NW = NC * NS
RPW = BATCH // NW
NBUF = 4

_MESH = plsc.VectorSubcoreMesh(core_axis_name="c", subcore_axis_name="s")


def _run_pipeline(n, mk_a, mk_b):
    def stat(i):
        j = i % NBUF
        mk_a(i, j).wait()
        if i >= 2:
            mk_b(0, (j + 2) % NBUF).wait()
        if i + 2 < n:
            mk_a(i + 2, (i + 2) % NBUF).start()
        mk_b(i, j).start()

    mk_a(0, 0).start()
    mk_a(1, 1).start()
    for i in range(NBUF):
        stat(i)

    t_hi = ((n - 2) // NBUF) * NBUF

    def step(ts, carry):
        for j in range(NBUF):
            i = ts * NBUF + j
            mk_a(0, j).wait()
            mk_b(0, (j + 2) % NBUF).wait()
            mk_a(i + 2, (i + 2) % NBUF).start()
            mk_b(i, j).start()
        return carry

    lax.fori_loop(1, t_hi // NBUF, step, 0)
    for i in range(t_hi, n):
        stat(i)
    mk_b(0, (n - 2) % NBUF).wait()
    mk_b(0, (n - 1) % NBUF).wait()


@functools.partial(
    pl.kernel,
    mesh=_MESH,
    out_type=jax.ShapeDtypeStruct((BATCH, HIST, DP), jnp.float32),
    scratch_types=[
        pltpu.VMEM((RPW * HIST,), jnp.int32),
        pltpu.VMEM((NBUF, HIST, DP), jnp.float32),
        pltpu.SemaphoreType.DMA((NBUF,)),
        pltpu.SemaphoreType.DMA((NBUF,)),
    ],
    compiler_params=pltpu.CompilerParams(
        use_tc_tiling_on_sc=True, needs_layout_passes=False),
)
def _gather_kernel(idx_hbm, table_hbm, out_hbm, idx_v, rows_v, sg, so):
    wid = lax.axis_index("s") * NC + lax.axis_index("c")
    b0 = wid * RPW

    pltpu.sync_copy(idx_hbm.at[pl.ds(b0 * HIST, RPW * HIST)], idx_v)

    def gather(i, b):
        return pltpu.make_async_copy(
            table_hbm.at[idx_v.at[pl.ds(i * HIST, HIST)]], rows_v.at[b],
            sg.at[b])

    def store(i, b):
        return pltpu.make_async_copy(
            rows_v.at[b], out_hbm.at[b0 + i], so.at[b])

    _run_pipeline(RPW, gather, store)


def kernel(indices, table):
    idx_flat = indices.reshape(-1)
    table_p = jnp.pad(table, ((0, 0), (0, DP - D)))
    out_p = _gather_kernel(idx_flat, table_p)
    return out_p[..., :D]
